# baseline (speedup 1.0000x reference)
import functools

import jax
import jax.numpy as jnp
from jax import lax
from jax.experimental import pallas as pl
from jax.experimental.pallas import tpu as pltpu
from jax.experimental.pallas import tpu_sc as plsc

BATCH = 16384
EMBED_DIM = 32
NUM_ROWS = 1000000

_info = plsc.get_sparse_core_info()
_NC, _NS = _info.num_cores, _info.num_subcores
_NW = _NC * _NS
_B_PER_W = BATCH // _NW


def _build_flat_idx(base_idx_v, flat_idx_v):
    def per_dim(c, _):
        off = c * NUM_ROWS
        for k in range(_B_PER_W // 16):
            chunk = base_idx_v[pl.ds(16 * k, 16)]
            flat_idx_v[pl.ds(c * _B_PER_W + 16 * k, 16)] = chunk + off
        return 0
    lax.fori_loop(0, EMBED_DIM, per_dim, 0)


def _gather_body(users_hbm, items_hbm, ut_hbm, it_hbm,
                 ug_hbm, ig_hbm,
                 uidx_v, iidx_v, fidx_u, fidx_i, urows_v, irows_v, usem, isem):
    wid = lax.axis_index("s") * _NC + lax.axis_index("c")
    base = wid * _B_PER_W
    pltpu.sync_copy(users_hbm.at[pl.ds(base, _B_PER_W)], uidx_v)
    pltpu.sync_copy(items_hbm.at[pl.ds(base, _B_PER_W)], iidx_v)
    _build_flat_idx(uidx_v, fidx_u)
    ucp = pltpu.async_copy(ut_hbm.at[fidx_u], urows_v, usem)
    _build_flat_idx(iidx_v, fidx_i)
    icp = pltpu.async_copy(it_hbm.at[fidx_i], irows_v, isem)
    ucp.wait()
    icp.wait()
    for c in range(EMBED_DIM):
        pltpu.sync_copy(urows_v.at[pl.ds(c * _B_PER_W, _B_PER_W)],
                        ug_hbm.at[c, pl.ds(base, _B_PER_W)])
        pltpu.sync_copy(irows_v.at[pl.ds(c * _B_PER_W, _B_PER_W)],
                        ig_hbm.at[c, pl.ds(base, _B_PER_W)])


def _sc_gather(users, items, ut_flat, it_flat):
    mesh = plsc.VectorSubcoreMesh(core_axis_name="c", subcore_axis_name="s")
    fn = pl.kernel(
        _gather_body,
        mesh=mesh,
        compiler_params=pltpu.CompilerParams(use_tc_tiling_on_sc=False),
        out_type=(
            jax.ShapeDtypeStruct((EMBED_DIM, BATCH), jnp.float32),
            jax.ShapeDtypeStruct((EMBED_DIM, BATCH), jnp.float32),
        ),
        scratch_types=[
            pltpu.VMEM((_B_PER_W,), jnp.int32),
            pltpu.VMEM((_B_PER_W,), jnp.int32),
            pltpu.VMEM((EMBED_DIM * _B_PER_W,), jnp.int32),
            pltpu.VMEM((EMBED_DIM * _B_PER_W,), jnp.int32),
            pltpu.VMEM((EMBED_DIM * _B_PER_W,), jnp.float32),
            pltpu.VMEM((EMBED_DIM * _B_PER_W,), jnp.float32),
            pltpu.SemaphoreType.DMA,
            pltpu.SemaphoreType.DMA,
        ],
    )
    return fn(users, items, ut_flat, it_flat)


def _dense_body(ug_ref, ig_ref, wu_ref, bu_ref, wi_ref, bi_ref, out_ref):
    uv = lax.dot_general(
        wu_ref[...], ug_ref[...],
        dimension_numbers=(((1,), (0,)), ((), ())),
        preferred_element_type=jnp.float32,
        precision=lax.Precision.HIGHEST,
    ) + bu_ref[...][:, None]
    iv = lax.dot_general(
        wi_ref[...], ig_ref[...],
        dimension_numbers=(((1,), (0,)), ((), ())),
        preferred_element_type=jnp.float32,
        precision=lax.Precision.HIGHEST,
    ) + bi_ref[...][:, None]
    out_ref[...] = jnp.sum(uv * iv, axis=0)


_TC_BLOCK = 4096


def _tc_dense(ug, ig, W_user, b_user, W_item, b_item):
    nblk = BATCH // _TC_BLOCK
    return pl.pallas_call(
        _dense_body,
        grid=(nblk,),
        in_specs=[
            pl.BlockSpec((EMBED_DIM, _TC_BLOCK), lambda i: (0, i)),
            pl.BlockSpec((EMBED_DIM, _TC_BLOCK), lambda i: (0, i)),
            pl.BlockSpec((EMBED_DIM, EMBED_DIM), lambda i: (0, 0)),
            pl.BlockSpec((EMBED_DIM,), lambda i: (0,)),
            pl.BlockSpec((EMBED_DIM, EMBED_DIM), lambda i: (0, 0)),
            pl.BlockSpec((EMBED_DIM,), lambda i: (0,)),
        ],
        out_specs=pl.BlockSpec((_TC_BLOCK,), lambda i: (i,)),
        out_shape=jax.ShapeDtypeStruct((BATCH,), jnp.float32),
    )(ug, ig, W_user, b_user, W_item, b_item)


@jax.jit
def kernel(users, items, user_embedding, item_embedding,
           W_user, b_user, W_item, b_item):
    users = users.astype(jnp.int32)
    items = items.astype(jnp.int32)
    ut_flat = user_embedding.T.reshape(-1)
    it_flat = item_embedding.T.reshape(-1)
    ug, ig = _sc_gather(users, items, ut_flat, it_flat)
    return _tc_dense(ug, ig, W_user, b_user, W_item, b_item)


# speedup vs baseline: 22.4681x; 22.4681x over previous
import functools

import jax
import jax.numpy as jnp
from jax import lax
from jax.experimental import pallas as pl
from jax.experimental.pallas import tpu as pltpu
from jax.experimental.pallas import tpu_sc as plsc

BATCH = 16384
EMBED_DIM = 32
NUM_ROWS = 1000000

_info = plsc.get_sparse_core_info()
_NC, _NS = _info.num_cores, _info.num_subcores
_NW = _NC * _NS
_B_PER_W = BATCH // _NW


_PADROW = 1 << 20
_CH = 1 << 17
_NCHUNK = _PADROW // _CH


def _detile_body(t_blk, flat_ref, sems):
    i = pl.program_id(0)
    for c in range(EMBED_DIM):
        pltpu.make_async_copy(
            t_blk.at[c], flat_ref.at[pl.ds(c * _PADROW + i * _CH, _CH)],
            sems.at[c],
        ).start()
    for c in range(EMBED_DIM):
        pltpu.make_async_copy(
            t_blk.at[c], flat_ref.at[pl.ds(c * _PADROW + i * _CH, _CH)],
            sems.at[c],
        ).wait()


def _tc_detile(t):
    return pl.pallas_call(
        _detile_body,
        grid=(_NCHUNK,),
        in_specs=[pl.BlockSpec((EMBED_DIM, _CH), lambda i: (0, i))],
        out_specs=pl.BlockSpec(memory_space=pl.ANY),
        out_shape=jax.ShapeDtypeStruct((EMBED_DIM * _PADROW,), jnp.float32),
        scratch_shapes=[pltpu.SemaphoreType.DMA((EMBED_DIM,))],
    )(t)


def _build_flat_idx(base_idx_v, flat_idx_v):
    def per_dim(c, _):
        off = c * _PADROW
        for k in range(_B_PER_W // 16):
            chunk = base_idx_v[pl.ds(16 * k, 16)]
            flat_idx_v[pl.ds(c * _B_PER_W + 16 * k, 16)] = chunk + off
        return 0
    lax.fori_loop(0, EMBED_DIM, per_dim, 0)


def _gather_body(idx_hbm, flat_hbm, out_hbm, bidx_v, fidx_v, rows_v, sem):
    wid = lax.axis_index("s") * _NC + lax.axis_index("c")
    base = wid * _B_PER_W
    pltpu.sync_copy(idx_hbm.at[pl.ds(base, _B_PER_W)], bidx_v)
    _build_flat_idx(bidx_v, fidx_v)
    pltpu.async_copy(flat_hbm.at[fidx_v], rows_v, sem).wait()
    for c in range(EMBED_DIM):
        pltpu.sync_copy(rows_v.at[pl.ds(c * _B_PER_W, _B_PER_W)],
                        out_hbm.at[c, pl.ds(base, _B_PER_W)])


def _sc_gather(idx, flat):
    mesh = plsc.VectorSubcoreMesh(core_axis_name="c", subcore_axis_name="s")
    fn = pl.kernel(
        _gather_body,
        mesh=mesh,
        compiler_params=pltpu.CompilerParams(use_tc_tiling_on_sc=False),
        out_type=jax.ShapeDtypeStruct((EMBED_DIM, BATCH), jnp.float32),
        scratch_types=[
            pltpu.VMEM((_B_PER_W,), jnp.int32),
            pltpu.VMEM((EMBED_DIM * _B_PER_W,), jnp.int32),
            pltpu.VMEM((EMBED_DIM * _B_PER_W,), jnp.float32),
            pltpu.SemaphoreType.DMA,
        ],
    )
    return fn(idx, flat)


def _dense_body(ug_ref, ig_ref, wu_ref, bu_ref, wi_ref, bi_ref, out_ref):
    uv = lax.dot_general(
        wu_ref[...], ug_ref[...],
        dimension_numbers=(((1,), (0,)), ((), ())),
        preferred_element_type=jnp.float32,
        precision=lax.Precision.HIGHEST,
    ) + bu_ref[...][:, None]
    iv = lax.dot_general(
        wi_ref[...], ig_ref[...],
        dimension_numbers=(((1,), (0,)), ((), ())),
        preferred_element_type=jnp.float32,
        precision=lax.Precision.HIGHEST,
    ) + bi_ref[...][:, None]
    out_ref[...] = jnp.sum(uv * iv, axis=0)


_TC_BLOCK = 4096


def _tc_dense(ug, ig, W_user, b_user, W_item, b_item):
    nblk = BATCH // _TC_BLOCK
    return pl.pallas_call(
        _dense_body,
        grid=(nblk,),
        in_specs=[
            pl.BlockSpec((EMBED_DIM, _TC_BLOCK), lambda i: (0, i)),
            pl.BlockSpec((EMBED_DIM, _TC_BLOCK), lambda i: (0, i)),
            pl.BlockSpec((EMBED_DIM, EMBED_DIM), lambda i: (0, 0)),
            pl.BlockSpec((EMBED_DIM,), lambda i: (0,)),
            pl.BlockSpec((EMBED_DIM, EMBED_DIM), lambda i: (0, 0)),
            pl.BlockSpec((EMBED_DIM,), lambda i: (0,)),
        ],
        out_specs=pl.BlockSpec((_TC_BLOCK,), lambda i: (i,)),
        out_shape=jax.ShapeDtypeStruct((BATCH,), jnp.float32),
    )(ug, ig, W_user, b_user, W_item, b_item)


@jax.jit
def kernel(users, items, user_embedding, item_embedding,
           W_user, b_user, W_item, b_item):
    users = users.astype(jnp.int32)
    items = items.astype(jnp.int32)
    uflat = _tc_detile(user_embedding.T)
    ug = _sc_gather(users, uflat)
    iflat = _tc_detile(item_embedding.T)
    ig = _sc_gather(items, iflat)
    return _tc_dense(ug, ig, W_user, b_user, W_item, b_item)


# speedup vs baseline: 28.5342x; 1.2700x over previous
import jax
import jax.numpy as jnp
from jax import lax
from jax.experimental import pallas as pl
from jax.experimental.pallas import tpu as pltpu
from jax.experimental.pallas import tpu_sc as plsc

BATCH = 16384
EMBED_DIM = 32
NUM_ROWS = 1000000
_NGRP = EMBED_DIM // 2

_info = plsc.get_sparse_core_info()
_NC, _NS = _info.num_cores, _info.num_subcores
_NW = _NC * _NS
_B_PER_W = BATCH // _NW

_PADROW = 1 << 20
_CH = 1 << 16
_NCHUNK = _PADROW // _CH


def _detile_body(t_blk, flat_ref, w_v, sems):
    i = pl.program_id(0)
    y = t_blk[...].astype(jnp.bfloat16)
    lo = lax.bitcast_convert_type(y[:_NGRP], jnp.uint16).astype(jnp.uint32)
    hi = lax.bitcast_convert_type(y[_NGRP:], jnp.uint16).astype(jnp.uint32)
    w_v[...] = lo | (hi << 16)
    for g in range(_NGRP):
        pltpu.make_async_copy(
            w_v.at[g], flat_ref.at[pl.ds(g * _PADROW + i * _CH, _CH)],
            sems.at[g],
        ).start()
    for g in range(_NGRP):
        pltpu.make_async_copy(
            w_v.at[g], flat_ref.at[pl.ds(g * _PADROW + i * _CH, _CH)],
            sems.at[g],
        ).wait()


def _tc_detile(t):
    return pl.pallas_call(
        _detile_body,
        grid=(_NCHUNK,),
        in_specs=[pl.BlockSpec((EMBED_DIM, _CH), lambda i: (0, i))],
        out_specs=pl.BlockSpec(memory_space=pl.ANY),
        out_shape=jax.ShapeDtypeStruct((_NGRP * _PADROW,), jnp.uint32),
        scratch_shapes=[
            pltpu.VMEM((_NGRP, _CH), jnp.uint32),
            pltpu.SemaphoreType.DMA((_NGRP,)),
        ],
    )(t)


def _build_flat_idx(base_idx_v, flat_idx_v):
    def per_grp(g, _):
        off = g * _PADROW
        for k in range(_B_PER_W // 16):
            chunk = base_idx_v[pl.ds(16 * k, 16)]
            flat_idx_v[pl.ds(g * _B_PER_W + 16 * k, 16)] = chunk + off
        return 0
    lax.fori_loop(0, _NGRP, per_grp, 0)


def _gather_body(idx_hbm, flat_hbm, out_hbm, bidx_v, fidx_v, rows_v, sem):
    wid = lax.axis_index("s") * _NC + lax.axis_index("c")
    base = wid * _B_PER_W
    pltpu.sync_copy(idx_hbm.at[pl.ds(base, _B_PER_W)], bidx_v)
    _build_flat_idx(bidx_v, fidx_v)
    pltpu.async_copy(flat_hbm.at[fidx_v], rows_v, sem).wait()
    for g in range(_NGRP):
        pltpu.sync_copy(rows_v.at[pl.ds(g * _B_PER_W, _B_PER_W)],
                        out_hbm.at[g, pl.ds(base, _B_PER_W)])


def _sc_gather(idx, flat):
    mesh = plsc.VectorSubcoreMesh(core_axis_name="c", subcore_axis_name="s")
    fn = pl.kernel(
        _gather_body,
        mesh=mesh,
        compiler_params=pltpu.CompilerParams(use_tc_tiling_on_sc=False),
        out_type=jax.ShapeDtypeStruct((_NGRP, BATCH), jnp.uint32),
        scratch_types=[
            pltpu.VMEM((_B_PER_W,), jnp.int32),
            pltpu.VMEM((_NGRP * _B_PER_W,), jnp.int32),
            pltpu.VMEM((_NGRP * _B_PER_W,), jnp.uint32),
            pltpu.SemaphoreType.DMA,
        ],
    )
    return fn(idx, flat)


def _unpack(x_u32):
    lo = lax.bitcast_convert_type(x_u32 << 16, jnp.float32)
    hi = lax.bitcast_convert_type(x_u32 & jnp.uint32(0xFFFF0000), jnp.float32)
    return lo, hi


def _dense_body(ug_ref, ig_ref, wu_ref, bu_ref, wi_ref, bi_ref, out_ref):
    dn = (((1,), (0,)), ((), ()))
    u_lo, u_hi = _unpack(ug_ref[...])
    i_lo, i_hi = _unpack(ig_ref[...])
    wu = wu_ref[...]
    wi = wi_ref[...]
    uv = (
        lax.dot_general(wu[:, :_NGRP], u_lo, dimension_numbers=dn,
                        preferred_element_type=jnp.float32,
                        precision=lax.Precision.HIGHEST)
        + lax.dot_general(wu[:, _NGRP:], u_hi, dimension_numbers=dn,
                          preferred_element_type=jnp.float32,
                          precision=lax.Precision.HIGHEST)
        + bu_ref[...][:, None]
    )
    iv = (
        lax.dot_general(wi[:, :_NGRP], i_lo, dimension_numbers=dn,
                        preferred_element_type=jnp.float32,
                        precision=lax.Precision.HIGHEST)
        + lax.dot_general(wi[:, _NGRP:], i_hi, dimension_numbers=dn,
                          preferred_element_type=jnp.float32,
                          precision=lax.Precision.HIGHEST)
        + bi_ref[...][:, None]
    )
    out_ref[...] = jnp.sum(uv * iv, axis=0)


_TC_BLOCK = 4096


def _tc_dense(ug, ig, W_user, b_user, W_item, b_item):
    nblk = BATCH // _TC_BLOCK
    return pl.pallas_call(
        _dense_body,
        grid=(nblk,),
        in_specs=[
            pl.BlockSpec((_NGRP, _TC_BLOCK), lambda i: (0, i)),
            pl.BlockSpec((_NGRP, _TC_BLOCK), lambda i: (0, i)),
            pl.BlockSpec((EMBED_DIM, EMBED_DIM), lambda i: (0, 0)),
            pl.BlockSpec((EMBED_DIM,), lambda i: (0,)),
            pl.BlockSpec((EMBED_DIM, EMBED_DIM), lambda i: (0, 0)),
            pl.BlockSpec((EMBED_DIM,), lambda i: (0,)),
        ],
        out_specs=pl.BlockSpec((_TC_BLOCK,), lambda i: (i,)),
        out_shape=jax.ShapeDtypeStruct((BATCH,), jnp.float32),
    )(ug, ig, W_user, b_user, W_item, b_item)


@jax.jit
def kernel(users, items, user_embedding, item_embedding,
           W_user, b_user, W_item, b_item):
    users = users.astype(jnp.int32)
    items = items.astype(jnp.int32)
    uflat = _tc_detile(user_embedding.T)
    ug = _sc_gather(users, uflat)
    iflat = _tc_detile(item_embedding.T)
    ig = _sc_gather(items, iflat)
    return _tc_dense(ug, ig, W_user, b_user, W_item, b_item)


# speedup vs baseline: 29.6904x; 1.0405x over previous
import jax
import jax.numpy as jnp
from jax import lax
from jax.experimental import pallas as pl
from jax.experimental.pallas import tpu as pltpu
from jax.experimental.pallas import tpu_sc as plsc

BATCH = 16384
EMBED_DIM = 32
NUM_ROWS = 1000000
_NGRP = EMBED_DIM // 2

_info = plsc.get_sparse_core_info()
_NC, _NS = _info.num_cores, _info.num_subcores
_NW = _NC * _NS
_B_PER_W = BATCH // _NW

_PADROW = 1 << 20
_CH = 1 << 16
_NCHUNK = _PADROW // _CH


def _detile_body(t_blk, flat_ref, w_v, sems):
    i = pl.program_id(0)
    b = i % 2

    @pl.when(i > 0)
    def _wait_prev():
        for g in range(_NGRP):
            pltpu.make_async_copy(
                w_v.at[1 - b, g],
                flat_ref.at[pl.ds(g * _PADROW + (i - 1) * _CH, _CH)],
                sems.at[1 - b, g],
            ).wait()

    y = t_blk[...].astype(jnp.bfloat16)
    lo = lax.bitcast_convert_type(y[:_NGRP], jnp.uint16).astype(jnp.uint32)
    hi = lax.bitcast_convert_type(y[_NGRP:], jnp.uint16).astype(jnp.uint32)
    w_v[b] = lo | (hi << 16)
    for g in range(_NGRP):
        pltpu.make_async_copy(
            w_v.at[b, g], flat_ref.at[pl.ds(g * _PADROW + i * _CH, _CH)],
            sems.at[b, g],
        ).start()

    @pl.when(i == _NCHUNK - 1)
    def _wait_last():
        for g in range(_NGRP):
            pltpu.make_async_copy(
                w_v.at[b, g], flat_ref.at[pl.ds(g * _PADROW + i * _CH, _CH)],
                sems.at[b, g],
            ).wait()


def _tc_detile(t):
    return pl.pallas_call(
        _detile_body,
        grid=(_NCHUNK,),
        in_specs=[pl.BlockSpec((EMBED_DIM, _CH), lambda i: (0, i))],
        out_specs=pl.BlockSpec(memory_space=pl.ANY),
        out_shape=jax.ShapeDtypeStruct((_NGRP * _PADROW,), jnp.uint32),
        scratch_shapes=[
            pltpu.VMEM((2, _NGRP, _CH), jnp.uint32),
            pltpu.SemaphoreType.DMA((2, _NGRP)),
        ],
    )(t)


def _build_flat_idx(base_idx_v, flat_idx_v):
    def per_grp(g, _):
        off = g * _PADROW
        for k in range(_B_PER_W // 16):
            chunk = base_idx_v[pl.ds(16 * k, 16)]
            flat_idx_v[pl.ds(g * _B_PER_W + 16 * k, 16)] = chunk + off
        return 0
    lax.fori_loop(0, _NGRP, per_grp, 0)


def _gather_body(idx_hbm, flat_hbm, out_hbm, bidx_v, fidx_v, rows_v, sem):
    wid = lax.axis_index("s") * _NC + lax.axis_index("c")
    base = wid * _B_PER_W
    pltpu.sync_copy(idx_hbm.at[pl.ds(base, _B_PER_W)], bidx_v)
    _build_flat_idx(bidx_v, fidx_v)
    pltpu.async_copy(flat_hbm.at[fidx_v], rows_v, sem).wait()
    for g in range(_NGRP):
        pltpu.sync_copy(rows_v.at[pl.ds(g * _B_PER_W, _B_PER_W)],
                        out_hbm.at[g, pl.ds(base, _B_PER_W)])


def _sc_gather(idx, flat):
    mesh = plsc.VectorSubcoreMesh(core_axis_name="c", subcore_axis_name="s")
    fn = pl.kernel(
        _gather_body,
        mesh=mesh,
        compiler_params=pltpu.CompilerParams(use_tc_tiling_on_sc=False),
        out_type=jax.ShapeDtypeStruct((_NGRP, BATCH), jnp.uint32),
        scratch_types=[
            pltpu.VMEM((_B_PER_W,), jnp.int32),
            pltpu.VMEM((_NGRP * _B_PER_W,), jnp.int32),
            pltpu.VMEM((_NGRP * _B_PER_W,), jnp.uint32),
            pltpu.SemaphoreType.DMA,
        ],
    )
    return fn(idx, flat)


def _unpack(x_u32):
    lo = lax.bitcast_convert_type(x_u32 << 16, jnp.float32)
    hi = lax.bitcast_convert_type(x_u32 & jnp.uint32(0xFFFF0000), jnp.float32)
    return lo, hi


def _dense_body(ug_ref, ig_ref, wu_ref, bu_ref, wi_ref, bi_ref, out_ref):
    dn = (((1,), (0,)), ((), ()))
    u_lo, u_hi = _unpack(ug_ref[...])
    i_lo, i_hi = _unpack(ig_ref[...])
    wu = wu_ref[...]
    wi = wi_ref[...]
    uv = (
        lax.dot_general(wu[:, :_NGRP], u_lo, dimension_numbers=dn,
                        preferred_element_type=jnp.float32,
                        precision=lax.Precision.HIGHEST)
        + lax.dot_general(wu[:, _NGRP:], u_hi, dimension_numbers=dn,
                          preferred_element_type=jnp.float32,
                          precision=lax.Precision.HIGHEST)
        + bu_ref[...][:, None]
    )
    iv = (
        lax.dot_general(wi[:, :_NGRP], i_lo, dimension_numbers=dn,
                        preferred_element_type=jnp.float32,
                        precision=lax.Precision.HIGHEST)
        + lax.dot_general(wi[:, _NGRP:], i_hi, dimension_numbers=dn,
                          preferred_element_type=jnp.float32,
                          precision=lax.Precision.HIGHEST)
        + bi_ref[...][:, None]
    )
    out_ref[...] = jnp.sum(uv * iv, axis=0)


_TC_BLOCK = 4096


def _tc_dense(ug, ig, W_user, b_user, W_item, b_item):
    nblk = BATCH // _TC_BLOCK
    return pl.pallas_call(
        _dense_body,
        grid=(nblk,),
        in_specs=[
            pl.BlockSpec((_NGRP, _TC_BLOCK), lambda i: (0, i)),
            pl.BlockSpec((_NGRP, _TC_BLOCK), lambda i: (0, i)),
            pl.BlockSpec((EMBED_DIM, EMBED_DIM), lambda i: (0, 0)),
            pl.BlockSpec((EMBED_DIM,), lambda i: (0,)),
            pl.BlockSpec((EMBED_DIM, EMBED_DIM), lambda i: (0, 0)),
            pl.BlockSpec((EMBED_DIM,), lambda i: (0,)),
        ],
        out_specs=pl.BlockSpec((_TC_BLOCK,), lambda i: (i,)),
        out_shape=jax.ShapeDtypeStruct((BATCH,), jnp.float32),
    )(ug, ig, W_user, b_user, W_item, b_item)


@jax.jit
def kernel(users, items, user_embedding, item_embedding,
           W_user, b_user, W_item, b_item):
    users = users.astype(jnp.int32)
    items = items.astype(jnp.int32)
    uflat = _tc_detile(user_embedding.T)
    ug = _sc_gather(users, uflat)
    iflat = _tc_detile(item_embedding.T)
    ig = _sc_gather(items, iflat)
    return _tc_dense(ug, ig, W_user, b_user, W_item, b_item)


# speedup vs baseline: 30.0279x; 1.0114x over previous
import jax
import jax.numpy as jnp
from jax import lax
from jax.experimental import pallas as pl
from jax.experimental.pallas import tpu as pltpu
from jax.experimental.pallas import tpu_sc as plsc

BATCH = 16384
EMBED_DIM = 32
NUM_ROWS = 1000000
_NGRP = EMBED_DIM // 2

_info = plsc.get_sparse_core_info()
_NC, _NS = _info.num_cores, _info.num_subcores
_NW = _NC * _NS
_B_PER_W = BATCH // _NW

_PADROW = 1 << 20
_CH = 1 << 16
_NCHUNK = _PADROW // _CH


def _detile_body(t_blk, flat_ref, w_v, sems):
    i = pl.program_id(0)
    b = i % 2

    @pl.when(i > 0)
    def _wait_prev():
        for g in range(_NGRP):
            pltpu.make_async_copy(
                w_v.at[1 - b, g],
                flat_ref.at[pl.ds(g * _PADROW + (i - 1) * _CH, _CH)],
                sems.at[1 - b, g],
            ).wait()

    y = t_blk[...].astype(jnp.bfloat16)
    lo = lax.bitcast_convert_type(y[:_NGRP], jnp.uint16).astype(jnp.uint32)
    hi = lax.bitcast_convert_type(y[_NGRP:], jnp.uint16).astype(jnp.uint32)
    w_v[b] = lo | (hi << 16)
    for g in range(_NGRP):
        pltpu.make_async_copy(
            w_v.at[b, g], flat_ref.at[pl.ds(g * _PADROW + i * _CH, _CH)],
            sems.at[b, g],
        ).start()

    @pl.when(i == _NCHUNK - 1)
    def _wait_last():
        for g in range(_NGRP):
            pltpu.make_async_copy(
                w_v.at[b, g], flat_ref.at[pl.ds(g * _PADROW + i * _CH, _CH)],
                sems.at[b, g],
            ).wait()


def _tc_detile(t):
    return pl.pallas_call(
        _detile_body,
        grid=(_NCHUNK,),
        in_specs=[pl.BlockSpec((EMBED_DIM, _CH), lambda i: (0, i))],
        out_specs=pl.BlockSpec(memory_space=pl.ANY),
        out_shape=jax.ShapeDtypeStruct((_NGRP * _PADROW,), jnp.uint32),
        scratch_shapes=[
            pltpu.VMEM((2, _NGRP, _CH), jnp.uint32),
            pltpu.SemaphoreType.DMA((2, _NGRP)),
        ],
    )(t)


def _build_flat_idx(base_idx_v, flat_idx_v):
    def per_grp(g, _):
        off = g * _PADROW
        for k in range(_B_PER_W // 16):
            chunk = base_idx_v[pl.ds(16 * k, 16)]
            flat_idx_v[pl.ds(g * _B_PER_W + 16 * k, 16)] = chunk + off
        return 0
    lax.fori_loop(0, _NGRP, per_grp, 0)


def _gather_body(idx_hbm, flat_hbm, out_hbm, bidx_v, fidx_v, rows_v, sem):
    wid = lax.axis_index("s") * _NC + lax.axis_index("c")
    base = wid * _B_PER_W
    pltpu.sync_copy(idx_hbm.at[pl.ds(base, _B_PER_W)], bidx_v)
    _build_flat_idx(bidx_v, fidx_v)
    pltpu.async_copy(flat_hbm.at[fidx_v], rows_v, sem).wait()
    for g in range(_NGRP):
        pltpu.sync_copy(rows_v.at[pl.ds(g * _B_PER_W, _B_PER_W)],
                        out_hbm.at[pl.ds(g * BATCH + base, _B_PER_W)])


def _sc_gather(idx, flat):
    mesh = plsc.VectorSubcoreMesh(core_axis_name="c", subcore_axis_name="s")
    fn = pl.kernel(
        _gather_body,
        mesh=mesh,
        compiler_params=pltpu.CompilerParams(use_tc_tiling_on_sc=False),
        out_type=jax.ShapeDtypeStruct((_NGRP * BATCH,), jnp.uint32),
        scratch_types=[
            pltpu.VMEM((_B_PER_W,), jnp.int32),
            pltpu.VMEM((_NGRP * _B_PER_W,), jnp.int32),
            pltpu.VMEM((_NGRP * _B_PER_W,), jnp.uint32),
            pltpu.SemaphoreType.DMA,
        ],
    )
    return fn(idx, flat)


def _unpack(x_u32):
    lo = lax.bitcast_convert_type(x_u32 << 16, jnp.float32)
    hi = lax.bitcast_convert_type(x_u32 & jnp.uint32(0xFFFF0000), jnp.float32)
    return lo, hi


def _dense_body(ug_ref, ig_ref, wu_ref, bu_ref, wi_ref, bi_ref, out_ref):
    dn = (((1,), (0,)), ((), ()))
    u_lo, u_hi = _unpack(ug_ref[...])
    i_lo, i_hi = _unpack(ig_ref[...])
    wu = wu_ref[...]
    wi = wi_ref[...]
    uv = (
        lax.dot_general(wu[:, :_NGRP], u_lo, dimension_numbers=dn,
                        preferred_element_type=jnp.float32,
                        precision=lax.Precision.HIGHEST)
        + lax.dot_general(wu[:, _NGRP:], u_hi, dimension_numbers=dn,
                          preferred_element_type=jnp.float32,
                          precision=lax.Precision.HIGHEST)
        + bu_ref[...][:, None, None]
    )
    iv = (
        lax.dot_general(wi[:, :_NGRP], i_lo, dimension_numbers=dn,
                        preferred_element_type=jnp.float32,
                        precision=lax.Precision.HIGHEST)
        + lax.dot_general(wi[:, _NGRP:], i_hi, dimension_numbers=dn,
                          preferred_element_type=jnp.float32,
                          precision=lax.Precision.HIGHEST)
        + bi_ref[...][:, None, None]
    )
    out_ref[...] = jnp.sum(uv * iv, axis=0)


_TC_BLOCK = 32


def _tc_dense(ug3, ig3, W_user, b_user, W_item, b_item):
    nblk = 128 // _TC_BLOCK
    return pl.pallas_call(
        _dense_body,
        grid=(nblk,),
        in_specs=[
            pl.BlockSpec((_NGRP, _TC_BLOCK, 128), lambda i: (0, i, 0)),
            pl.BlockSpec((_NGRP, _TC_BLOCK, 128), lambda i: (0, i, 0)),
            pl.BlockSpec((EMBED_DIM, EMBED_DIM), lambda i: (0, 0)),
            pl.BlockSpec((EMBED_DIM,), lambda i: (0,)),
            pl.BlockSpec((EMBED_DIM, EMBED_DIM), lambda i: (0, 0)),
            pl.BlockSpec((EMBED_DIM,), lambda i: (0,)),
        ],
        out_specs=pl.BlockSpec((_TC_BLOCK, 128), lambda i: (i, 0)),
        out_shape=jax.ShapeDtypeStruct((128, 128), jnp.float32),
    )(ug3, ig3, W_user, b_user, W_item, b_item)


@jax.jit
def kernel(users, items, user_embedding, item_embedding,
           W_user, b_user, W_item, b_item):
    users = users.astype(jnp.int32)
    items = items.astype(jnp.int32)
    uflat = _tc_detile(user_embedding.T)
    ug = _sc_gather(users, uflat).reshape(_NGRP, 128, 128)
    iflat = _tc_detile(item_embedding.T)
    ig = _sc_gather(items, iflat).reshape(_NGRP, 128, 128)
    out = _tc_dense(ug, ig, W_user, b_user, W_item, b_item)
    return out.reshape(BATCH)


# speedup vs baseline: 30.0399x; 1.0004x over previous
import jax
import jax.numpy as jnp
from jax import lax
from jax.experimental import pallas as pl
from jax.experimental.pallas import tpu as pltpu
from jax.experimental.pallas import tpu_sc as plsc

BATCH = 16384
EMBED_DIM = 32
NUM_ROWS = 1000000
_NGRP = EMBED_DIM // 2

_info = plsc.get_sparse_core_info()
_NC, _NS = _info.num_cores, _info.num_subcores
_NW = _NC * _NS
_B_PER_W = BATCH // _NW

_PADROW = 1 << 20
_CH = 1 << 16
_NCHUNK = _PADROW // _CH


def _detile_body(tlo_blk, thi_blk, flat_ref, w_v, sems):
    i = pl.program_id(0)
    b = i % 2

    @pl.when(i > 0)
    def _wait_prev():
        for g in range(_NGRP):
            pltpu.make_async_copy(
                w_v.at[1 - b, g],
                flat_ref.at[pl.ds(g * _PADROW + (i - 1) * _CH, _CH)],
                sems.at[1 - b, g],
            ).wait()

    lo = lax.bitcast_convert_type(
        tlo_blk[...].astype(jnp.bfloat16), jnp.uint16).astype(jnp.uint32)
    hi = lax.bitcast_convert_type(
        thi_blk[...].astype(jnp.bfloat16), jnp.uint16).astype(jnp.uint32)
    w_v[b] = lo | (hi << 16)
    for g in range(_NGRP):
        pltpu.make_async_copy(
            w_v.at[b, g], flat_ref.at[pl.ds(g * _PADROW + i * _CH, _CH)],
            sems.at[b, g],
        ).start()

    @pl.when(i == _NCHUNK - 1)
    def _wait_last():
        for g in range(_NGRP):
            pltpu.make_async_copy(
                w_v.at[b, g], flat_ref.at[pl.ds(g * _PADROW + i * _CH, _CH)],
                sems.at[b, g],
            ).wait()


def _tc_detile(t):
    return pl.pallas_call(
        _detile_body,
        grid=(_NCHUNK,),
        in_specs=[pl.BlockSpec((_NGRP, _CH), lambda i: (0, i)),
                  pl.BlockSpec((_NGRP, _CH), lambda i: (1, i))],
        out_specs=pl.BlockSpec(memory_space=pl.ANY),
        out_shape=jax.ShapeDtypeStruct((_NGRP * _PADROW,), jnp.uint32),
        scratch_shapes=[
            pltpu.VMEM((2, _NGRP, _CH), jnp.uint32),
            pltpu.SemaphoreType.DMA((2, _NGRP)),
        ],
    )(t, t)


def _build_flat_idx(base_idx_v, flat_idx_v):
    def per_grp(g, _):
        off = g * _PADROW
        for k in range(_B_PER_W // 16):
            chunk = base_idx_v[pl.ds(16 * k, 16)]
            flat_idx_v[pl.ds(g * _B_PER_W + 16 * k, 16)] = chunk + off
        return 0
    lax.fori_loop(0, _NGRP, per_grp, 0)


def _gather_body(idx_hbm, flat_hbm, out_hbm, bidx_v, fidx_v, rows_v, sem):
    wid = lax.axis_index("s") * _NC + lax.axis_index("c")
    base = wid * _B_PER_W
    pltpu.sync_copy(idx_hbm.at[pl.ds(base, _B_PER_W)], bidx_v)
    _build_flat_idx(bidx_v, fidx_v)
    pltpu.async_copy(flat_hbm.at[fidx_v], rows_v, sem).wait()
    for g in range(_NGRP):
        pltpu.sync_copy(rows_v.at[pl.ds(g * _B_PER_W, _B_PER_W)],
                        out_hbm.at[pl.ds(g * BATCH + base, _B_PER_W)])


def _sc_gather(idx, flat):
    mesh = plsc.VectorSubcoreMesh(core_axis_name="c", subcore_axis_name="s")
    fn = pl.kernel(
        _gather_body,
        mesh=mesh,
        compiler_params=pltpu.CompilerParams(use_tc_tiling_on_sc=False),
        out_type=jax.ShapeDtypeStruct((_NGRP * BATCH,), jnp.uint32),
        scratch_types=[
            pltpu.VMEM((_B_PER_W,), jnp.int32),
            pltpu.VMEM((_NGRP * _B_PER_W,), jnp.int32),
            pltpu.VMEM((_NGRP * _B_PER_W,), jnp.uint32),
            pltpu.SemaphoreType.DMA,
        ],
    )
    return fn(idx, flat)


def _unpack(x_u32):
    lo = lax.bitcast_convert_type(x_u32 << 16, jnp.float32)
    hi = lax.bitcast_convert_type(x_u32 & jnp.uint32(0xFFFF0000), jnp.float32)
    return lo, hi


def _dense_body(ug_ref, ig_ref, wu_ref, bu_ref, wi_ref, bi_ref, out_ref):
    dn = (((1,), (0,)), ((), ()))
    u_lo, u_hi = _unpack(ug_ref[...])
    i_lo, i_hi = _unpack(ig_ref[...])
    wu = wu_ref[...]
    wi = wi_ref[...]
    uv = (
        lax.dot_general(wu[:, :_NGRP], u_lo, dimension_numbers=dn,
                        preferred_element_type=jnp.float32,
                        precision=lax.Precision.HIGHEST)
        + lax.dot_general(wu[:, _NGRP:], u_hi, dimension_numbers=dn,
                          preferred_element_type=jnp.float32,
                          precision=lax.Precision.HIGHEST)
        + bu_ref[...][:, None, None]
    )
    iv = (
        lax.dot_general(wi[:, :_NGRP], i_lo, dimension_numbers=dn,
                        preferred_element_type=jnp.float32,
                        precision=lax.Precision.HIGHEST)
        + lax.dot_general(wi[:, _NGRP:], i_hi, dimension_numbers=dn,
                          preferred_element_type=jnp.float32,
                          precision=lax.Precision.HIGHEST)
        + bi_ref[...][:, None, None]
    )
    out_ref[...] = jnp.sum(uv * iv, axis=0)


_TC_BLOCK = 32


def _tc_dense(ug3, ig3, W_user, b_user, W_item, b_item):
    nblk = 128 // _TC_BLOCK
    return pl.pallas_call(
        _dense_body,
        grid=(nblk,),
        in_specs=[
            pl.BlockSpec((_NGRP, _TC_BLOCK, 128), lambda i: (0, i, 0)),
            pl.BlockSpec((_NGRP, _TC_BLOCK, 128), lambda i: (0, i, 0)),
            pl.BlockSpec((EMBED_DIM, EMBED_DIM), lambda i: (0, 0)),
            pl.BlockSpec((EMBED_DIM,), lambda i: (0,)),
            pl.BlockSpec((EMBED_DIM, EMBED_DIM), lambda i: (0, 0)),
            pl.BlockSpec((EMBED_DIM,), lambda i: (0,)),
        ],
        out_specs=pl.BlockSpec((_TC_BLOCK, 128), lambda i: (i, 0)),
        out_shape=jax.ShapeDtypeStruct((128, 128), jnp.float32),
    )(ug3, ig3, W_user, b_user, W_item, b_item)


@jax.jit
def kernel(users, items, user_embedding, item_embedding,
           W_user, b_user, W_item, b_item):
    users = users.astype(jnp.int32)
    items = items.astype(jnp.int32)
    uflat = _tc_detile(user_embedding.T)
    ug = _sc_gather(users, uflat).reshape(_NGRP, 128, 128)
    iflat = _tc_detile(item_embedding.T)
    ig = _sc_gather(items, iflat).reshape(_NGRP, 128, 128)
    out = _tc_dense(ug, ig, W_user, b_user, W_item, b_item)
    return out.reshape(BATCH)
